# Initial kernel scaffold; baseline (speedup 1.0000x reference)
import functools

import jax
import jax.numpy as jnp
from jax import lax
from jax.experimental import pallas as pl
from jax.experimental.pallas import tpu as pltpu
from jax.experimental.pallas import tpu_sc as plsc

N = 10000
E = 320000
F_IN = 128
H = 256
EMB = 128
G = 64
EPS = 1e-5
RS = float(1.0 / (1.0 + EPS) ** 0.5)

NC = 2
NS = 16
EB = 100
ROWS2D = E // EB
RPW = ROWS2D // (NC * NS)
RPS = ROWS2D // NS
RT = N // NS
ZR = 125
MB = 1000


def _deg_body(dst2d, deg0, deg1, onesb, zbuf, idxb, acc):
    c = lax.axis_index("c")
    s = lax.axis_index("s")
    w = s * NC + c

    def fill_ones(i, carry):
        onesb[i, :] = jnp.ones((16,), jnp.float32)
        return carry

    lax.fori_loop(0, EB, fill_ones, 0)

    def fill_z(i, carry):
        zbuf[i, :] = jnp.zeros((16,), jnp.float32)
        return carry

    lax.fori_loop(0, ZR, fill_z, 0)
    for k in range(RT // ZR):
        pltpu.sync_copy(zbuf, acc.at[pl.ds(s * RT + k * ZR, ZR)])
    plsc.subcore_barrier()

    def body(j, carry):
        pltpu.sync_copy(dst2d.at[w * RPW + j], idxb)
        pltpu.sync_copy(onesb, acc.at[idxb], add=True)
        return carry

    lax.fori_loop(0, RPW, body, 0)
    plsc.subcore_barrier()

    @pl.when(c == 0)
    def _():
        for k in range(RT // ZR):
            sl = pl.ds(s * RT + k * ZR, ZR)
            pltpu.sync_copy(acc.at[sl], deg0.at[sl])

    @pl.when(c == 1)
    def _():
        for k in range(RT // ZR):
            sl = pl.ds(s * RT + k * ZR, ZR)
            pltpu.sync_copy(acc.at[sl], deg1.at[sl])


_deg_call = pl.kernel(
    _deg_body,
    out_type=[jax.ShapeDtypeStruct((N, 16), jnp.float32)] * 2,
    mesh=plsc.VectorSubcoreMesh(core_axis_name="c", subcore_axis_name="s"),
    scratch_types=[
        pltpu.VMEM((EB, 16), jnp.float32),
        pltpu.VMEM((ZR, 16), jnp.float32),
        pltpu.VMEM((EB,), jnp.int32),
        pltpu.VMEM_SHARED((N, 16), jnp.float32),
    ],
)


def _agg_body(D, ga, gb, src2d, dst2d, outa, outb, srcb, dstb, rows, zbuf, acc,
              sem):
    c = lax.axis_index("c")
    s = lax.axis_index("s")

    def fill_z(i, carry):
        for t in range(D // 16):
            zbuf[i, pl.ds(t * 16, 16)] = jnp.zeros((16,), jnp.float32)
        return carry

    lax.fori_loop(0, ZR, fill_z, 0)
    for k in range(RT // ZR):
        pltpu.sync_copy(zbuf, acc.at[pl.ds(s * RT + k * ZR, ZR)])
    plsc.subcore_barrier()

    def edge_loop(g):
        def body(j, carry):
            r = s * RPS + j
            pltpu.sync_copy(src2d.at[r], srcb)
            pltpu.sync_copy(dst2d.at[r], dstb)
            pltpu.async_copy(g.at[srcb], rows, sem).wait()
            pltpu.sync_copy(rows, acc.at[dstb], add=True)
            return carry

        lax.fori_loop(0, RPS, body, 0)

    @pl.when(c == 0)
    def _():
        edge_loop(ga)

    @pl.when(c == 1)
    def _():
        edge_loop(gb)

    plsc.subcore_barrier()

    @pl.when(c == 0)
    def _():
        for k in range(RT // ZR):
            sl = pl.ds(s * RT + k * ZR, ZR)
            pltpu.sync_copy(acc.at[sl], outa.at[sl])

    @pl.when(c == 1)
    def _():
        for k in range(RT // ZR):
            sl = pl.ds(s * RT + k * ZR, ZR)
            pltpu.sync_copy(acc.at[sl], outb.at[sl])


def _make_agg(D):
    return pl.kernel(
        functools.partial(_agg_body, D),
        out_type=[jax.ShapeDtypeStruct((N, D), jnp.float32)] * 2,
        mesh=plsc.VectorSubcoreMesh(core_axis_name="c", subcore_axis_name="s"),
        scratch_types=[
            pltpu.VMEM((EB,), jnp.int32),
            pltpu.VMEM((EB,), jnp.int32),
            pltpu.VMEM((EB, D), jnp.float32),
            pltpu.VMEM((ZR, D), jnp.float32),
            pltpu.VMEM_SHARED((N, D), jnp.float32),
            pltpu.SemaphoreType.DMA,
        ],
    )


_agg128 = _make_agg(H // 2)
_agg64 = _make_agg(EMB // 2)


def _prep_body(x_ref, w1_ref, d0_ref, d1_ref, dinv_ref, ga_ref, gb_ref):
    deg = d0_ref[:, 0:1] + d1_ref[:, 0:1] + 1.0
    dinv = lax.rsqrt(deg)
    dinv_ref[...] = jnp.broadcast_to(dinv, (MB, 128))
    g = jnp.dot(x_ref[...], w1_ref[...], preferred_element_type=jnp.float32)
    g = g * dinv
    ga_ref[...] = g[:, : H // 2]
    gb_ref[...] = g[:, H // 2:]


_prep_call = pl.pallas_call(
    _prep_body,
    grid=(N // MB,),
    in_specs=[
        pl.BlockSpec((MB, F_IN), lambda i: (i, 0)),
        pl.BlockSpec((F_IN, H), lambda i: (0, 0)),
        pl.BlockSpec((MB, 16), lambda i: (i, 0)),
        pl.BlockSpec((MB, 16), lambda i: (i, 0)),
    ],
    out_specs=[
        pl.BlockSpec((MB, 128), lambda i: (i, 0)),
        pl.BlockSpec((MB, H // 2), lambda i: (i, 0)),
        pl.BlockSpec((MB, H // 2), lambda i: (i, 0)),
    ],
    out_shape=[jax.ShapeDtypeStruct((N, 128), jnp.float32)] * 3,
)


def _layer_body(Dout, sa, sb, ga, gb, dinv, w_ref, b_ref, gam_ref, bet_ref,
                oa, ob):
    dv = dinv[:, 0:1]
    s_full = jnp.concatenate([sa[...], sb[...]], axis=1)
    g_full = jnp.concatenate([ga[...], gb[...]], axis=1)
    hpre = dv * (s_full + g_full) + b_ref[...]
    h = jnp.maximum(hpre * (gam_ref[...] * RS) + bet_ref[...], 0.0)
    gnew = dv * jnp.dot(h, w_ref[...], preferred_element_type=jnp.float32)
    oa[...] = gnew[:, : Dout // 2]
    ob[...] = gnew[:, Dout // 2:]


def _make_layer(Dout):
    return pl.pallas_call(
        functools.partial(_layer_body, Dout),
        grid=(N // MB,),
        in_specs=[
            pl.BlockSpec((MB, H // 2), lambda i: (i, 0)),
            pl.BlockSpec((MB, H // 2), lambda i: (i, 0)),
            pl.BlockSpec((MB, H // 2), lambda i: (i, 0)),
            pl.BlockSpec((MB, H // 2), lambda i: (i, 0)),
            pl.BlockSpec((MB, 128), lambda i: (i, 0)),
            pl.BlockSpec((H, Dout), lambda i: (0, 0)),
            pl.BlockSpec((1, H), lambda i: (0, 0)),
            pl.BlockSpec((1, H), lambda i: (0, 0)),
            pl.BlockSpec((1, H), lambda i: (0, 0)),
        ],
        out_specs=[
            pl.BlockSpec((MB, Dout // 2), lambda i: (i, 0)),
            pl.BlockSpec((MB, Dout // 2), lambda i: (i, 0)),
        ],
        out_shape=[jax.ShapeDtypeStruct((N, Dout // 2), jnp.float32)] * 2,
    )


_layer_h = _make_layer(H)
_layer_e = _make_layer(EMB)


def _final_body(sa, sb, ga, gb, dinv, b3_ref, batr, batc, fw1, fb1, fg1, fbt1,
                fw2, fb2, fg2, fbt2, fw3, fb3, out_ref, mx_ref):
    dv = dinv[:, 0:1]
    s_full = jnp.concatenate([sa[...], sb[...]], axis=1)
    g_full = jnp.concatenate([ga[...], gb[...]], axis=1)
    h3 = dv * (s_full + g_full) + b3_ref[...]
    seg = lax.broadcasted_iota(jnp.int32, (G, N), 0)
    m = (seg == batr[...]).astype(jnp.float32)
    cnt = jnp.sum(m, axis=1, keepdims=True)
    sums = jnp.dot(m, h3, preferred_element_type=jnp.float32)
    mean = sums / jnp.maximum(cnt, 1.0)
    bc = batc[...]

    def seg_max(gi, carry):
        mask = bc == gi
        vals = jnp.where(mask, h3, -jnp.inf)
        mx_ref[pl.ds(gi, 1), :] = jnp.max(vals, axis=0, keepdims=True)
        return carry

    lax.fori_loop(0, G, seg_max, 0)
    mx = mx_ref[...]
    mx = jnp.where(mx == -jnp.inf, 0.0, mx)
    emb = jnp.concatenate([mean, mx], axis=1)
    o = jnp.dot(emb, fw1[...], preferred_element_type=jnp.float32) + fb1[...]
    o = jnp.maximum(o * (fg1[...] * RS) + fbt1[...], 0.0)
    o = jnp.dot(o, fw2[...], preferred_element_type=jnp.float32) + fb2[...]
    o = jnp.maximum(o * (fg2[...] * RS) + fbt2[...], 0.0)
    out_ref[...] = (
        jnp.dot(o, fw3[...], preferred_element_type=jnp.float32) + fb3[...]
    )


_final_call = pl.pallas_call(
    _final_body,
    out_shape=jax.ShapeDtypeStruct((G, 128), jnp.float32),
    scratch_shapes=[pltpu.VMEM((G, 128), jnp.float32)],
)


def kernel(x, edge_index, batch, W1, b1, g1, bt1, W2, b2, g2, bt2, W3, b3,
           fW1, fb1, fg1, fbt1, fW2, fb2, fg2, fbt2, fW3, fb3):
    src2d = edge_index[0].reshape(ROWS2D, EB)
    dst2d = edge_index[1].reshape(ROWS2D, EB)
    deg0, deg1 = _deg_call(dst2d)
    dinv, g1a, g1b = _prep_call(x, W1, deg0, deg1)
    s1a, s1b = _agg128(g1a, g1b, src2d, dst2d)
    g2a, g2b = _layer_h(s1a, s1b, g1a, g1b, dinv, W2,
                        b1.reshape(1, -1), g1.reshape(1, -1),
                        bt1.reshape(1, -1))
    s2a, s2b = _agg128(g2a, g2b, src2d, dst2d)
    g3a, g3b = _layer_e(s2a, s2b, g2a, g2b, dinv, W3,
                        b2.reshape(1, -1), g2.reshape(1, -1),
                        bt2.reshape(1, -1))
    s3a, s3b = _agg64(g3a, g3b, src2d, dst2d)
    batr = batch.reshape(1, N)
    batc = batch.reshape(N, 1)
    fW3p = jnp.concatenate([fW3, jnp.zeros((fW3.shape[0], 126), jnp.float32)],
                           axis=1)
    fb3p = jnp.concatenate([fb3, jnp.zeros((126,), jnp.float32)]).reshape(1, -1)
    outp = _final_call(s3a, s3b, g3a, g3b, dinv, b3.reshape(1, -1), batr, batc,
                       fW1, fb1.reshape(1, -1), fg1.reshape(1, -1),
                       fbt1.reshape(1, -1), fW2, fb2.reshape(1, -1),
                       fg2.reshape(1, -1), fbt2.reshape(1, -1), fW3p, fb3p)
    return outp[:, :2]


# speedup vs baseline: 8.8387x; 8.8387x over previous
import functools

import jax
import jax.numpy as jnp
from jax import lax
from jax.experimental import pallas as pl
from jax.experimental.pallas import tpu as pltpu
from jax.experimental.pallas import tpu_sc as plsc

N = 10000
NPAD = 10240
E = 320000
F_IN = 128
H = 256
EMB = 128
G = 64
EPS = 1e-5
RS = float(1.0 / (1.0 + EPS) ** 0.5)

NC = 2
NS = 16
EB = 100
ROWS2D = E // EB
RPW = ROWS2D // (NC * NS)
RPS = ROWS2D // NS
RT = NPAD // NS
ZR = 128
MB = 1000


def _deg_body(dst2d, deg0, deg1, onesb, zbuf, idxb, acc):
    c = lax.axis_index("c")
    s = lax.axis_index("s")
    w = s * NC + c

    def fill_ones(i, carry):
        for t in range(8):
            onesb[i, pl.ds(t * 16, 16)] = jnp.ones((16,), jnp.float32)
        return carry

    lax.fori_loop(0, EB, fill_ones, 0)

    def fill_z(i, carry):
        for t in range(8):
            zbuf[i, pl.ds(t * 16, 16)] = jnp.zeros((16,), jnp.float32)
        return carry

    lax.fori_loop(0, ZR, fill_z, 0)
    for k in range(RT // ZR):
        pltpu.sync_copy(zbuf, acc.at[pl.ds(s * RT + k * ZR, ZR)])
    plsc.subcore_barrier()

    def body(j, carry):
        pltpu.sync_copy(dst2d.at[w * RPW + j], idxb)
        pltpu.sync_copy(onesb, acc.at[idxb.at[0]], add=True)
        return carry

    lax.fori_loop(0, RPW, body, 0)
    plsc.subcore_barrier()

    @pl.when(c == 0)
    def _():
        for k in range(RT // ZR):
            sl = pl.ds(s * RT + k * ZR, ZR)
            pltpu.sync_copy(acc.at[sl], deg0.at[sl])

    @pl.when(c == 1)
    def _():
        for k in range(RT // ZR):
            sl = pl.ds(s * RT + k * ZR, ZR)
            pltpu.sync_copy(acc.at[sl], deg1.at[sl])


_deg_call = pl.kernel(
    _deg_body,
    out_type=[jax.ShapeDtypeStruct((NPAD, 128), jnp.float32)] * 2,
    mesh=plsc.VectorSubcoreMesh(core_axis_name="c", subcore_axis_name="s"),
    scratch_types=[
        pltpu.VMEM((EB, 128), jnp.float32),
        pltpu.VMEM((ZR, 128), jnp.float32),
        pltpu.VMEM((1, EB), jnp.int32),
        pltpu.VMEM_SHARED((NPAD, 128), jnp.float32),
    ],
)


def _agg_body(D, ga, gb, src2d, dst2d, outa, outb, srcb, dstb, rows, zbuf, acc,
              sem):
    c = lax.axis_index("c")
    s = lax.axis_index("s")

    def fill_z(i, carry):
        for t in range(D // 16):
            zbuf[i, pl.ds(t * 16, 16)] = jnp.zeros((16,), jnp.float32)
        return carry

    lax.fori_loop(0, ZR, fill_z, 0)
    for k in range(RT // ZR):
        pltpu.sync_copy(zbuf, acc.at[pl.ds(s * RT + k * ZR, ZR)])
    plsc.subcore_barrier()

    def edge_loop(g):
        def body(j, carry):
            r = s * RPS + j
            pltpu.sync_copy(src2d.at[r], srcb)
            pltpu.sync_copy(dst2d.at[r], dstb)
            pltpu.async_copy(g.at[srcb.at[0]], rows, sem).wait()
            pltpu.sync_copy(rows, acc.at[dstb.at[0]], add=True)
            return carry

        lax.fori_loop(0, RPS, body, 0)

    @pl.when(c == 0)
    def _():
        edge_loop(ga)

    @pl.when(c == 1)
    def _():
        edge_loop(gb)

    plsc.subcore_barrier()

    @pl.when(c == 0)
    def _():
        for k in range(RT // ZR):
            sl = pl.ds(s * RT + k * ZR, ZR)
            pltpu.sync_copy(acc.at[sl], outa.at[sl])

    @pl.when(c == 1)
    def _():
        for k in range(RT // ZR):
            sl = pl.ds(s * RT + k * ZR, ZR)
            pltpu.sync_copy(acc.at[sl], outb.at[sl])


def _make_agg(D):
    return pl.kernel(
        functools.partial(_agg_body, D),
        out_type=[jax.ShapeDtypeStruct((NPAD, D), jnp.float32)] * 2,
        mesh=plsc.VectorSubcoreMesh(core_axis_name="c", subcore_axis_name="s"),
        scratch_types=[
            pltpu.VMEM((1, EB), jnp.int32),
            pltpu.VMEM((1, EB), jnp.int32),
            pltpu.VMEM((EB, D), jnp.float32),
            pltpu.VMEM((ZR, D), jnp.float32),
            pltpu.VMEM_SHARED((NPAD, D), jnp.float32),
            pltpu.SemaphoreType.DMA,
        ],
    )


_agg128 = _make_agg(H // 2)


def _aggsplit_body(g, src2d, dst2d, outa, outb, srcb, dstb, rows, zbuf, acc,
                   sem):
    c = lax.axis_index("c")
    s = lax.axis_index("s")
    w = s * NC + c

    def fill_z(i, carry):
        for t in range(EMB // 16):
            zbuf[i, pl.ds(t * 16, 16)] = jnp.zeros((16,), jnp.float32)
        return carry

    lax.fori_loop(0, ZR, fill_z, 0)
    for k in range(RT // ZR):
        pltpu.sync_copy(zbuf, acc.at[pl.ds(s * RT + k * ZR, ZR)])
    plsc.subcore_barrier()

    def body(j, carry):
        r = w * RPW + j
        pltpu.sync_copy(src2d.at[r], srcb)
        pltpu.sync_copy(dst2d.at[r], dstb)
        pltpu.async_copy(g.at[srcb.at[0]], rows, sem).wait()
        pltpu.sync_copy(rows, acc.at[dstb.at[0]], add=True)
        return carry

    lax.fori_loop(0, RPW, body, 0)
    plsc.subcore_barrier()

    @pl.when(c == 0)
    def _():
        for k in range(RT // ZR):
            sl = pl.ds(s * RT + k * ZR, ZR)
            pltpu.sync_copy(acc.at[sl], outa.at[sl])

    @pl.when(c == 1)
    def _():
        for k in range(RT // ZR):
            sl = pl.ds(s * RT + k * ZR, ZR)
            pltpu.sync_copy(acc.at[sl], outb.at[sl])


_aggsplit = pl.kernel(
    _aggsplit_body,
    out_type=[jax.ShapeDtypeStruct((NPAD, EMB), jnp.float32)] * 2,
    mesh=plsc.VectorSubcoreMesh(core_axis_name="c", subcore_axis_name="s"),
    scratch_types=[
        pltpu.VMEM((1, EB), jnp.int32),
        pltpu.VMEM((1, EB), jnp.int32),
        pltpu.VMEM((EB, EMB), jnp.float32),
        pltpu.VMEM((ZR, EMB), jnp.float32),
        pltpu.VMEM_SHARED((NPAD, EMB), jnp.float32),
        pltpu.SemaphoreType.DMA,
    ],
)


def _prep_body(x_ref, w1_ref, d0_ref, d1_ref, dinv_ref, ga_ref, gb_ref):
    deg = d0_ref[:, 0:1] + d1_ref[:, 0:1] + 1.0
    dinv = lax.rsqrt(deg)
    dinv_ref[...] = jnp.broadcast_to(dinv, (MB, 128))
    g = jnp.dot(x_ref[...], w1_ref[...], preferred_element_type=jnp.float32)
    g = g * dinv
    ga_ref[...] = g[:, : H // 2]
    gb_ref[...] = g[:, H // 2:]


_prep_call = pl.pallas_call(
    _prep_body,
    grid=(N // MB,),
    in_specs=[
        pl.BlockSpec((MB, F_IN), lambda i: (i, 0)),
        pl.BlockSpec((F_IN, H), lambda i: (0, 0)),
        pl.BlockSpec((MB, 128), lambda i: (i, 0)),
        pl.BlockSpec((MB, 128), lambda i: (i, 0)),
    ],
    out_specs=[
        pl.BlockSpec((MB, 128), lambda i: (i, 0)),
        pl.BlockSpec((MB, H // 2), lambda i: (i, 0)),
        pl.BlockSpec((MB, H // 2), lambda i: (i, 0)),
    ],
    out_shape=[jax.ShapeDtypeStruct((N, 128), jnp.float32)] * 3,
)


def _layer_body(Dout, split, sa, sb, ga, gb, dinv, w_ref, b_ref, gam_ref,
                bet_ref, *outs):
    dv = dinv[:, 0:1]
    s_full = jnp.concatenate([sa[...], sb[...]], axis=1)
    g_full = jnp.concatenate([ga[...], gb[...]], axis=1)
    hpre = dv * (s_full + g_full) + b_ref[...]
    h = jnp.maximum(hpre * (gam_ref[...] * RS) + bet_ref[...], 0.0)
    gnew = dv * jnp.dot(h, w_ref[...], preferred_element_type=jnp.float32)
    if split:
        outs[0][...] = gnew[:, : Dout // 2]
        outs[1][...] = gnew[:, Dout // 2:]
    else:
        outs[0][...] = gnew


def _make_layer(Dout, split):
    return pl.pallas_call(
        functools.partial(_layer_body, Dout, split),
        grid=(N // MB,),
        in_specs=[
            pl.BlockSpec((MB, H // 2), lambda i: (i, 0)),
            pl.BlockSpec((MB, H // 2), lambda i: (i, 0)),
            pl.BlockSpec((MB, H // 2), lambda i: (i, 0)),
            pl.BlockSpec((MB, H // 2), lambda i: (i, 0)),
            pl.BlockSpec((MB, 128), lambda i: (i, 0)),
            pl.BlockSpec((H, Dout), lambda i: (0, 0)),
            pl.BlockSpec((1, H), lambda i: (0, 0)),
            pl.BlockSpec((1, H), lambda i: (0, 0)),
            pl.BlockSpec((1, H), lambda i: (0, 0)),
        ],
        out_specs=[
            pl.BlockSpec((MB, Dout // 2 if split else Dout), lambda i: (i, 0))
        ] * (2 if split else 1),
        out_shape=[jax.ShapeDtypeStruct((N, Dout // 2 if split else Dout),
                                        jnp.float32)] * (2 if split else 1),
    )


_layer_h = _make_layer(H, True)
_layer_e = _make_layer(EMB, False)


def _final_body(sa, sb, g3, dinv, b3_ref, batr, batc, fw1, fb1, fg1, fbt1,
                fw2, fb2, fg2, fbt2, fw3, fb3, out_ref, mx_ref):
    dv = dinv[:, 0:1]
    h3 = dv * (sa[...] + sb[...] + g3[...]) + b3_ref[...]
    seg = lax.broadcasted_iota(jnp.int32, (G, N), 0)
    m = (seg == batr[...]).astype(jnp.float32)
    cnt = jnp.sum(m, axis=1, keepdims=True)
    sums = jnp.dot(m, h3, preferred_element_type=jnp.float32)
    mean = sums / jnp.maximum(cnt, 1.0)
    bc = batc[...]

    def seg_max(gi, carry):
        mask = bc == gi
        vals = jnp.where(mask, h3, -jnp.inf)
        mx_ref[pl.ds(gi, 1), :] = jnp.max(vals, axis=0, keepdims=True)
        return carry

    lax.fori_loop(0, G, seg_max, 0)
    mx = mx_ref[...]
    mx = jnp.where(mx == -jnp.inf, 0.0, mx)
    emb = jnp.concatenate([mean, mx], axis=1)
    o = jnp.dot(emb, fw1[...], preferred_element_type=jnp.float32) + fb1[...]
    o = jnp.maximum(o * (fg1[...] * RS) + fbt1[...], 0.0)
    o = jnp.dot(o, fw2[...], preferred_element_type=jnp.float32) + fb2[...]
    o = jnp.maximum(o * (fg2[...] * RS) + fbt2[...], 0.0)
    out_ref[...] = (
        jnp.dot(o, fw3[...], preferred_element_type=jnp.float32) + fb3[...]
    )


_final_call = pl.pallas_call(
    _final_body,
    out_shape=jax.ShapeDtypeStruct((G, 128), jnp.float32),
    scratch_shapes=[pltpu.VMEM((G, 128), jnp.float32)],
)


def kernel(x, edge_index, batch, W1, b1, g1, bt1, W2, b2, g2, bt2, W3, b3,
           fW1, fb1, fg1, fbt1, fW2, fb2, fg2, fbt2, fW3, fb3):
    src2d = edge_index[0].reshape(ROWS2D, 1, EB)
    dst2d = edge_index[1].reshape(ROWS2D, 1, EB)
    deg0, deg1 = _deg_call(dst2d)
    dinv, g1a, g1b = _prep_call(x, W1, deg0, deg1)
    s1a, s1b = _agg128(g1a, g1b, src2d, dst2d)
    g2a, g2b = _layer_h(s1a, s1b, g1a, g1b, dinv, W2,
                        b1.reshape(1, -1), g1.reshape(1, -1),
                        bt1.reshape(1, -1))
    s2a, s2b = _agg128(g2a, g2b, src2d, dst2d)
    (g3,) = _layer_e(s2a, s2b, g2a, g2b, dinv, W3,
                     b2.reshape(1, -1), g2.reshape(1, -1),
                     bt2.reshape(1, -1))
    s3a, s3b = _aggsplit(g3, src2d, dst2d)
    s3a, s3b = s3a[:N], s3b[:N]
    batr = batch.reshape(1, N)
    batc = batch.reshape(N, 1)
    fW3p = jnp.concatenate([fW3, jnp.zeros((fW3.shape[0], 126), jnp.float32)],
                           axis=1)
    fb3p = jnp.concatenate([fb3, jnp.zeros((126,), jnp.float32)]).reshape(1, -1)
    outp = _final_call(s3a, s3b, g3, dinv, b3.reshape(1, -1), batr, batc,
                       fW1, fb1.reshape(1, -1), fg1.reshape(1, -1),
                       fbt1.reshape(1, -1), fW2, fb2.reshape(1, -1),
                       fg2.reshape(1, -1), fbt2.reshape(1, -1), fW3p, fb3p)
    return outp[:, :2]


# speedup vs baseline: 18.5345x; 2.0970x over previous
import functools

import jax
import jax.numpy as jnp
from jax import lax
from jax.experimental import pallas as pl
from jax.experimental.pallas import tpu as pltpu
from jax.experimental.pallas import tpu_sc as plsc

N = 10000
NPAD = 10240
E = 320000
F_IN = 128
H = 256
EMB = 128
G = 64
EPS = 1e-5
RS = float(1.0 / (1.0 + EPS) ** 0.5)

NC = 2
NS = 16
EB = 100
ROWS2D = E // EB
RPW = ROWS2D // (NC * NS)
RPS = ROWS2D // NS
SECR = 50
RT = NPAD // NS
ZB = 64
MB = 1000


def _zero_fill(buf, D):
    def fill(i, carry):
        for t in range(D // 16):
            buf[i, pl.ds(t * 16, 16)] = jnp.zeros((16,), jnp.float32)
        return carry

    lax.fori_loop(0, ZB, fill, 0)


def _zero_acc(buf, acc, s, semz):
    for k in range(RT // ZB):
        pltpu.async_copy(buf.at[pl.ds(0, ZB)],
                         acc.at[pl.ds(s * RT + k * ZB, ZB)], semz)
    for k in range(RT // ZB):
        pltpu.make_async_copy(buf.at[pl.ds(0, ZB)],
                              acc.at[pl.ds(s * RT + k * ZB, ZB)], semz).wait()


def _writeout(acc, out, s):
    for k in range(RT // ZB):
        sl = pl.ds(s * RT + k * ZB, ZB)
        pltpu.sync_copy(acc.at[sl], out.at[sl])


def _gather_scatter(g, src2d, dst2d, acc, srcc, dstc, rows0, rows1, sem0,
                    sem1, base, nsec):
    for sec in range(nsec):
        b = base + sec * SECR
        pltpu.sync_copy(src2d.at[pl.ds(b, SECR)], srcc)
        pltpu.sync_copy(dst2d.at[pl.ds(b, SECR)], dstc)
        pltpu.async_copy(g.at[srcc.at[0, 0]], rows0, sem0)
        pltpu.async_copy(g.at[srcc.at[1, 0]], rows1, sem1)

        def body(i, carry):
            j = 2 * i
            pltpu.make_async_copy(g.at[srcc.at[j, 0]], rows0, sem0).wait()
            pltpu.sync_copy(rows0, acc.at[dstc.at[j, 0]], add=True)

            @pl.when(j + 2 < SECR)
            def _():
                pltpu.async_copy(g.at[srcc.at[j + 2, 0]], rows0, sem0)

            pltpu.make_async_copy(g.at[srcc.at[j + 1, 0]], rows1, sem1).wait()
            pltpu.sync_copy(rows1, acc.at[dstc.at[j + 1, 0]], add=True)

            @pl.when(j + 3 < SECR)
            def _():
                pltpu.async_copy(g.at[srcc.at[j + 3, 0]], rows1, sem1)

            return carry

        lax.fori_loop(0, SECR // 2, body, 0)


def _deg_body(dst2d, deg0, deg1, onesb, zbuf, idxb, acc):
    c = lax.axis_index("c")
    s = lax.axis_index("s")
    w = s * NC + c

    def fill_ones(i, carry):
        for t in range(8):
            onesb[i, pl.ds(t * 16, 16)] = jnp.ones((16,), jnp.float32)
        return carry

    lax.fori_loop(0, EB, fill_ones, 0)
    _zero_fill(zbuf, 128)
    for k in range(RT // ZB):
        pltpu.sync_copy(zbuf.at[pl.ds(0, ZB)],
                        acc.at[pl.ds(s * RT + k * ZB, ZB)])
    plsc.subcore_barrier()

    pltpu.sync_copy(dst2d.at[pl.ds(w * RPW, RPW)], idxb)

    def body(j, carry):
        pltpu.sync_copy(onesb, acc.at[idxb.at[j, 0]], add=True)
        return carry

    lax.fori_loop(0, RPW, body, 0)
    plsc.subcore_barrier()

    @pl.when(c == 0)
    def _():
        _writeout(acc, deg0, s)

    @pl.when(c == 1)
    def _():
        _writeout(acc, deg1, s)


_deg_call = pl.kernel(
    _deg_body,
    out_type=[jax.ShapeDtypeStruct((NPAD, 128), jnp.float32)] * 2,
    mesh=plsc.VectorSubcoreMesh(core_axis_name="c", subcore_axis_name="s"),
    scratch_types=[
        pltpu.VMEM((EB, 128), jnp.float32),
        pltpu.VMEM((ZB, 128), jnp.float32),
        pltpu.VMEM((RPW, 1, EB), jnp.int32),
        pltpu.VMEM_SHARED((NPAD, 128), jnp.float32),
    ],
)


def _agg_body(D, ga, gb, src2d, dst2d, outa, outb, srcc, dstc, rows0, rows1,
              acc, sem0, sem1, semz):
    c = lax.axis_index("c")
    s = lax.axis_index("s")

    _zero_fill(rows0, D)
    _zero_acc(rows0, acc, s, semz)
    plsc.subcore_barrier()

    @pl.when(c == 0)
    def _():
        _gather_scatter(ga, src2d, dst2d, acc, srcc, dstc, rows0, rows1,
                        sem0, sem1, s * RPS, RPS // SECR)

    @pl.when(c == 1)
    def _():
        _gather_scatter(gb, src2d, dst2d, acc, srcc, dstc, rows0, rows1,
                        sem0, sem1, s * RPS, RPS // SECR)

    plsc.subcore_barrier()

    @pl.when(c == 0)
    def _():
        _writeout(acc, outa, s)

    @pl.when(c == 1)
    def _():
        _writeout(acc, outb, s)


def _make_agg(D):
    return pl.kernel(
        functools.partial(_agg_body, D),
        out_type=[jax.ShapeDtypeStruct((NPAD, D), jnp.float32)] * 2,
        mesh=plsc.VectorSubcoreMesh(core_axis_name="c", subcore_axis_name="s"),
        scratch_types=[
            pltpu.VMEM((SECR, 1, EB), jnp.int32),
            pltpu.VMEM((SECR, 1, EB), jnp.int32),
            pltpu.VMEM((EB, D), jnp.float32),
            pltpu.VMEM((EB, D), jnp.float32),
            pltpu.VMEM_SHARED((NPAD, D), jnp.float32),
            pltpu.SemaphoreType.DMA,
            pltpu.SemaphoreType.DMA,
            pltpu.SemaphoreType.DMA,
        ],
    )


_agg128 = _make_agg(H // 2)


def _aggsplit_body(g, src2d, dst2d, outa, outb, srcc, dstc, rows0, rows1,
                   acc, sem0, sem1, semz):
    c = lax.axis_index("c")
    s = lax.axis_index("s")
    w = s * NC + c

    _zero_fill(rows0, EMB)
    _zero_acc(rows0, acc, s, semz)
    plsc.subcore_barrier()

    _gather_scatter(g, src2d, dst2d, acc, srcc, dstc, rows0, rows1, sem0,
                    sem1, w * RPW, RPW // SECR)

    plsc.subcore_barrier()

    @pl.when(c == 0)
    def _():
        _writeout(acc, outa, s)

    @pl.when(c == 1)
    def _():
        _writeout(acc, outb, s)


_aggsplit = pl.kernel(
    _aggsplit_body,
    out_type=[jax.ShapeDtypeStruct((NPAD, EMB), jnp.float32)] * 2,
    mesh=plsc.VectorSubcoreMesh(core_axis_name="c", subcore_axis_name="s"),
    scratch_types=[
        pltpu.VMEM((SECR, 1, EB), jnp.int32),
        pltpu.VMEM((SECR, 1, EB), jnp.int32),
        pltpu.VMEM((EB, EMB), jnp.float32),
        pltpu.VMEM((EB, EMB), jnp.float32),
        pltpu.VMEM_SHARED((NPAD, EMB), jnp.float32),
        pltpu.SemaphoreType.DMA,
        pltpu.SemaphoreType.DMA,
        pltpu.SemaphoreType.DMA,
    ],
)


def _prep_body(x_ref, w1_ref, d0_ref, d1_ref, dinv_ref, ga_ref, gb_ref):
    deg = d0_ref[:, 0:1] + d1_ref[:, 0:1] + 1.0
    dinv = lax.rsqrt(deg)
    dinv_ref[...] = jnp.broadcast_to(dinv, (MB, 128))
    g = jnp.dot(x_ref[...], w1_ref[...], preferred_element_type=jnp.float32)
    g = g * dinv
    ga_ref[...] = g[:, : H // 2]
    gb_ref[...] = g[:, H // 2:]


_prep_call = pl.pallas_call(
    _prep_body,
    grid=(N // MB,),
    in_specs=[
        pl.BlockSpec((MB, F_IN), lambda i: (i, 0)),
        pl.BlockSpec((F_IN, H), lambda i: (0, 0)),
        pl.BlockSpec((MB, 128), lambda i: (i, 0)),
        pl.BlockSpec((MB, 128), lambda i: (i, 0)),
    ],
    out_specs=[
        pl.BlockSpec((MB, 128), lambda i: (i, 0)),
        pl.BlockSpec((MB, H // 2), lambda i: (i, 0)),
        pl.BlockSpec((MB, H // 2), lambda i: (i, 0)),
    ],
    out_shape=[jax.ShapeDtypeStruct((N, 128), jnp.float32)] * 3,
)


def _layer_body(Dout, split, sa, sb, ga, gb, dinv, w_ref, b_ref, gam_ref,
                bet_ref, *outs):
    dv = dinv[:, 0:1]
    s_full = jnp.concatenate([sa[...], sb[...]], axis=1)
    g_full = jnp.concatenate([ga[...], gb[...]], axis=1)
    hpre = dv * (s_full + g_full) + b_ref[...]
    h = jnp.maximum(hpre * (gam_ref[...] * RS) + bet_ref[...], 0.0)
    gnew = dv * jnp.dot(h, w_ref[...], preferred_element_type=jnp.float32)
    if split:
        outs[0][...] = gnew[:, : Dout // 2]
        outs[1][...] = gnew[:, Dout // 2:]
    else:
        outs[0][...] = gnew


def _make_layer(Dout, split):
    return pl.pallas_call(
        functools.partial(_layer_body, Dout, split),
        grid=(N // MB,),
        in_specs=[
            pl.BlockSpec((MB, H // 2), lambda i: (i, 0)),
            pl.BlockSpec((MB, H // 2), lambda i: (i, 0)),
            pl.BlockSpec((MB, H // 2), lambda i: (i, 0)),
            pl.BlockSpec((MB, H // 2), lambda i: (i, 0)),
            pl.BlockSpec((MB, 128), lambda i: (i, 0)),
            pl.BlockSpec((H, Dout), lambda i: (0, 0)),
            pl.BlockSpec((1, H), lambda i: (0, 0)),
            pl.BlockSpec((1, H), lambda i: (0, 0)),
            pl.BlockSpec((1, H), lambda i: (0, 0)),
        ],
        out_specs=[
            pl.BlockSpec((MB, Dout // 2 if split else Dout), lambda i: (i, 0))
        ] * (2 if split else 1),
        out_shape=[jax.ShapeDtypeStruct((N, Dout // 2 if split else Dout),
                                        jnp.float32)] * (2 if split else 1),
    )


_layer_h = _make_layer(H, True)
_layer_e = _make_layer(EMB, False)


def _final_body(sa, sb, g3, dinv, b3_ref, batr, batc, fw1, fb1, fg1, fbt1,
                fw2, fb2, fg2, fbt2, fw3, fb3, out_ref, mx_ref):
    dv = dinv[:, 0:1]
    h3 = dv * (sa[...] + sb[...] + g3[...]) + b3_ref[...]
    seg = lax.broadcasted_iota(jnp.int32, (G, N), 0)
    m = (seg == batr[...]).astype(jnp.float32)
    cnt = jnp.sum(m, axis=1, keepdims=True)
    sums = jnp.dot(m, h3, preferred_element_type=jnp.float32)
    mean = sums / jnp.maximum(cnt, 1.0)
    bc = batc[...]

    def seg_max(gi, carry):
        mask = bc == gi
        vals = jnp.where(mask, h3, -jnp.inf)
        mx_ref[pl.ds(gi, 1), :] = jnp.max(vals, axis=0, keepdims=True)
        return carry

    lax.fori_loop(0, G, seg_max, 0)
    mx = mx_ref[...]
    mx = jnp.where(mx == -jnp.inf, 0.0, mx)
    emb = jnp.concatenate([mean, mx], axis=1)
    o = jnp.dot(emb, fw1[...], preferred_element_type=jnp.float32) + fb1[...]
    o = jnp.maximum(o * (fg1[...] * RS) + fbt1[...], 0.0)
    o = jnp.dot(o, fw2[...], preferred_element_type=jnp.float32) + fb2[...]
    o = jnp.maximum(o * (fg2[...] * RS) + fbt2[...], 0.0)
    out_ref[...] = (
        jnp.dot(o, fw3[...], preferred_element_type=jnp.float32) + fb3[...]
    )


_final_call = pl.pallas_call(
    _final_body,
    out_shape=jax.ShapeDtypeStruct((G, 128), jnp.float32),
    scratch_shapes=[pltpu.VMEM((G, 128), jnp.float32)],
)


def kernel(x, edge_index, batch, W1, b1, g1, bt1, W2, b2, g2, bt2, W3, b3,
           fW1, fb1, fg1, fbt1, fW2, fb2, fg2, fbt2, fW3, fb3):
    src2d = edge_index[0].reshape(ROWS2D, 1, EB)
    dst2d = edge_index[1].reshape(ROWS2D, 1, EB)
    deg0, deg1 = _deg_call(dst2d)
    dinv, g1a, g1b = _prep_call(x, W1, deg0, deg1)
    s1a, s1b = _agg128(g1a, g1b, src2d, dst2d)
    g2a, g2b = _layer_h(s1a, s1b, g1a, g1b, dinv, W2,
                        b1.reshape(1, -1), g1.reshape(1, -1),
                        bt1.reshape(1, -1))
    s2a, s2b = _agg128(g2a, g2b, src2d, dst2d)
    (g3,) = _layer_e(s2a, s2b, g2a, g2b, dinv, W3,
                     b2.reshape(1, -1), g2.reshape(1, -1),
                     bt2.reshape(1, -1))
    s3a, s3b = _aggsplit(g3, src2d, dst2d)
    s3a, s3b = s3a[:N], s3b[:N]
    batr = batch.reshape(1, N)
    batc = batch.reshape(N, 1)
    fW3p = jnp.concatenate([fW3, jnp.zeros((fW3.shape[0], 126), jnp.float32)],
                           axis=1)
    fb3p = jnp.concatenate([fb3, jnp.zeros((126,), jnp.float32)]).reshape(1, -1)
    outp = _final_call(s3a, s3b, g3, dinv, b3.reshape(1, -1), batr, batc,
                       fW1, fb1.reshape(1, -1), fg1.reshape(1, -1),
                       fbt1.reshape(1, -1), fW2, fb2.reshape(1, -1),
                       fg2.reshape(1, -1), fbt2.reshape(1, -1), fW3p, fb3p)
    return outp[:, :2]
